# baseline (speedup 1.0000x reference)
import jax
import jax.numpy as jnp
from jax import lax
from jax.experimental import pallas as pl
from jax.experimental.pallas import tpu as pltpu
from jax.experimental.pallas import tpu_sc as plsc

_B, _C, _L, _N = 16, 512, 4096, 1024
_GAMMA = 1.0
_CW = 16
_HC = 4
_NQ = _CW // _HC
_LANES = 16
_NG = _N // _LANES
_NSUB = 16
_CCORE = _NSUB * _CW


def _sampler_body(inp, loc_in, out, loc_all, i0_v, w1_v, inb0, inb1,
                  outb, tmp, mrg, shm, sem0, sem1):
    cid = lax.axis_index("c")
    sid = lax.axis_index("s")
    wid = cid * _NSUB + sid
    c0 = wid * _CW

    def run_idx_loop(b):
        @plsc.parallel_loop(0, _NG, unroll=2)
        def idx_body(j):
            loc = loc_all[pl.ds(b * _N + j * _LANES, _LANES)]
            i0 = loc.astype(jnp.int32)
            i0_v[pl.ds(j * _LANES, _LANES)] = i0
            w1_v[pl.ds(j * _LANES, _LANES)] = loc - i0.astype(jnp.float32)

    def compute_quarter(buf, q):
        @plsc.parallel_loop(0, _NG, unroll=8)
        def grp_body(g):
            n_base = g * _LANES
            sl = pl.ds(n_base, _LANES)
            i0 = i0_v[sl]
            w1 = w1_v[sl]
            i1 = jnp.minimum(i0 + 1, _L - 1)
            w0 = 1.0 - w1
            for c in range(_HC):
                c_idx = jnp.full((_LANES,), c, jnp.int32)
                v0 = plsc.load_gather(buf, [c_idx, i0])
                v1 = plsc.load_gather(buf, [c_idx, i1])
                outb[q * _HC + c, sl] = w0 * v0 + w1 * v1

    def in_slab(b, q):
        return inp.at[b, pl.ds(c0 + q * _HC, _HC)]

    bufs = (inb0, inb1)
    sems = (sem0, sem1)

    pltpu.sync_copy(loc_in, loc_all)
    pltpu.async_copy(in_slab(0, 0), inb0, sem0)

    r_slab = sid // 2
    wh = sid % 2

    def per_batch(b, _):
        with jax.named_scope("idx_phase"):
            run_idx_loop(b)

        for q in range(_NQ):
            if q + 1 < _NQ:
                pltpu.async_copy(in_slab(b, q + 1),
                                 bufs[(q + 1) % 2], sems[(q + 1) % 2])
            else:
                @pl.when(b + 1 < _B)
                def _():
                    pltpu.async_copy(in_slab(b + 1, 0), bufs[0], sems[0])

            with jax.named_scope("in_wait"):
                pltpu.make_async_copy(in_slab(b, q), bufs[q % 2],
                                      sems[q % 2]).wait()
            with jax.named_scope("gather"):
                compute_quarter(bufs[q % 2], q)

        with jax.named_scope("mrg_put"):
            pltpu.sync_copy(outb, shm.at[sid])
        with jax.named_scope("mrg_bar1"):
            plsc.subcore_barrier()
        with jax.named_scope("mrg_get"):
            pltpu.sync_copy(
                shm.at[pl.ds(wh * 8, 8), :, pl.ds(r_slab * 128, 128)], tmp)
        with jax.named_scope("mrg_ilv"):
            @plsc.parallel_loop(0, 128, unroll=2)
            def col_body(m):
                j = m // _CW
                c = m - j * _CW
                m_idx = jnp.full((_LANES,), m, jnp.int32)
                ni = lax.iota(jnp.int32, _LANES)
                for k in range(8):
                    v = tmp[j, c, pl.ds(k * _LANES, _LANES)]
                    plsc.store_scatter(mrg, [k * _LANES + ni, m_idx], v)
        with jax.named_scope("out_dma"):
            pltpu.sync_copy(
                mrg,
                out.at[b, pl.ds(r_slab * 128, 128),
                       pl.ds(cid * _CCORE + wh * 128, 128)],
            )
        with jax.named_scope("mrg_bar2"):
            plsc.subcore_barrier()
        return 0

    lax.fori_loop(0, _B, per_batch, 0)


def kernel(input, point, offset):
    loc = jnp.clip(point[:, :, 0] + _GAMMA * offset[:, :, 0], 0.0,
                   float(_L - 1)).reshape(_B * _N)
    mesh = plsc.VectorSubcoreMesh(core_axis_name="c", subcore_axis_name="s")
    f = pl.kernel(
        _sampler_body,
        out_type=jax.ShapeDtypeStruct((_B, _N, _C), jnp.float32),
        mesh=mesh,
        scratch_types=[
            pltpu.VMEM((_B * _N,), jnp.float32),
            pltpu.VMEM((_N,), jnp.int32),
            pltpu.VMEM((_N,), jnp.float32),
            pltpu.VMEM((_HC, _L), jnp.float32),
            pltpu.VMEM((_HC, _L), jnp.float32),
            pltpu.VMEM((_CW, _N), jnp.float32),
            pltpu.VMEM((8, _CW, 128), jnp.float32),
            pltpu.VMEM((128, 128), jnp.float32),
            pltpu.VMEM_SHARED((_NSUB, _CW, _N), jnp.float32),
            pltpu.SemaphoreType.DMA,
            pltpu.SemaphoreType.DMA,
        ],
        compiler_params=pltpu.CompilerParams(needs_layout_passes=False),
    )
    return f(input, loc)


# speedup vs baseline: 1.5257x; 1.5257x over previous
import jax
import jax.numpy as jnp
from jax import lax
from jax.experimental import pallas as pl
from jax.experimental.pallas import tpu as pltpu
from jax.experimental.pallas import tpu_sc as plsc

_B, _C, _L, _N = 16, 512, 4096, 1024
_GAMMA = 1.0
_CW = 16
_HC = 4
_NQ = _CW // _HC
_LANES = 16
_NG = _N // _LANES
_NSUB = 16
_ROWS = _N // _NSUB
_CCORE = _NSUB * _CW


def _sampler_body(inp, loc_in, out, loc_all, i0_v, w1_v, inb0, inb1,
                  outb, tmp, mrg, shm, sem0, sem1, semo):
    cid = lax.axis_index("c")
    sid = lax.axis_index("s")
    wid = cid * _NSUB + sid
    c0 = wid * _CW

    def run_idx_loop(b):
        @plsc.parallel_loop(0, _NG, unroll=2)
        def idx_body(j):
            loc = loc_all[pl.ds(b * _N + j * _LANES, _LANES)]
            i0 = loc.astype(jnp.int32)
            i0_v[pl.ds(j * _LANES, _LANES)] = i0
            w1_v[pl.ds(j * _LANES, _LANES)] = loc - i0.astype(jnp.float32)

    def compute_quarter(buf, q):
        @plsc.parallel_loop(0, _NG, unroll=8)
        def grp_body(g):
            n_base = g * _LANES
            sl = pl.ds(n_base, _LANES)
            i0 = i0_v[sl]
            w1 = w1_v[sl]
            i1 = jnp.minimum(i0 + 1, _L - 1)
            w0 = 1.0 - w1
            s_idx = jnp.full((_LANES,), g // 4, jnp.int32)
            p_base = ((n_base % 64) + lax.iota(jnp.int32, _LANES)) * _CW + q * _HC
            for c in range(_HC):
                c_idx = jnp.full((_LANES,), c, jnp.int32)
                v0 = plsc.load_gather(buf, [c_idx, i0])
                v1 = plsc.load_gather(buf, [c_idx, i1])
                r = w0 * v0 + w1 * v1
                p = p_base + c
                plsc.store_scatter(outb, [s_idx, p >> 7, p & 127], r)

    def in_slab(b, q):
        return inp.at[b, pl.ds(c0 + q * _HC, _HC)]

    bufs = (inb0, inb1)
    sems = (sem0, sem1)

    pltpu.sync_copy(loc_in, loc_all)
    pltpu.async_copy(in_slab(0, 0), inb0, sem0)

    def out_slab(b):
        return out.at[b, pl.ds(sid * _ROWS, _ROWS),
                      pl.ds(cid * _CCORE, _CCORE)]

    def per_batch(b, _):
        with jax.named_scope("idx_phase"):
            run_idx_loop(b)

        for q in range(_NQ):
            if q + 1 < _NQ:
                pltpu.async_copy(in_slab(b, q + 1),
                                 bufs[(q + 1) % 2], sems[(q + 1) % 2])
            else:
                @pl.when(b + 1 < _B)
                def _():
                    pltpu.async_copy(in_slab(b + 1, 0), bufs[0], sems[0])

            with jax.named_scope("in_wait"):
                pltpu.make_async_copy(in_slab(b, q), bufs[q % 2],
                                      sems[q % 2]).wait()
            with jax.named_scope("gather"):
                compute_quarter(bufs[q % 2], q)

        with jax.named_scope("mrg_put"):
            pltpu.sync_copy(outb, shm.at[sid])
        with jax.named_scope("mrg_bar1"):
            plsc.subcore_barrier()
        with jax.named_scope("out_wait"):
            @pl.when(b > 0)
            def _():
                pltpu.make_async_copy(mrg, out_slab(b - 1), semo).wait()
        with jax.named_scope("mrg_get"):
            pltpu.sync_copy(shm.at[:, sid], tmp)
        with jax.named_scope("mrg_ilv"):
            @plsc.parallel_loop(0, _ROWS, unroll=2)
            def row_body(n):
                po = (n % 8) * _CW
                ph = n // 8
                for j in range(_NSUB):
                    v = tmp[j, ph, pl.ds(po, _LANES)]
                    plsc.store_scatter(
                        mrg, [jnp.full((_LANES,), n, jnp.int32),
                              j * _CW + lax.iota(jnp.int32, _LANES)], v)
        with jax.named_scope("out_dma"):
            pltpu.async_copy(mrg, out_slab(b), semo)
        with jax.named_scope("mrg_bar2"):
            plsc.subcore_barrier()
        return 0

    lax.fori_loop(0, _B, per_batch, 0)
    pltpu.make_async_copy(mrg, out_slab(_B - 1), semo).wait()


def kernel(input, point, offset):
    loc = jnp.clip(point[:, :, 0] + _GAMMA * offset[:, :, 0], 0.0,
                   float(_L - 1)).reshape(_B * _N)
    mesh = plsc.VectorSubcoreMesh(core_axis_name="c", subcore_axis_name="s")
    f = pl.kernel(
        _sampler_body,
        out_type=jax.ShapeDtypeStruct((_B, _N, _C), jnp.float32),
        mesh=mesh,
        scratch_types=[
            pltpu.VMEM((_B * _N,), jnp.float32),
            pltpu.VMEM((_N,), jnp.int32),
            pltpu.VMEM((_N,), jnp.float32),
            pltpu.VMEM((_HC, _L), jnp.float32),
            pltpu.VMEM((_HC, _L), jnp.float32),
            pltpu.VMEM((_NSUB, 8, 128), jnp.float32),
            pltpu.VMEM((_NSUB, 8, 128), jnp.float32),
            pltpu.VMEM((_ROWS, _CCORE), jnp.float32),
            pltpu.VMEM_SHARED((_NSUB, _NSUB, 8, 128), jnp.float32),
            pltpu.SemaphoreType.DMA,
            pltpu.SemaphoreType.DMA,
            pltpu.SemaphoreType.DMA,
        ],
        compiler_params=pltpu.CompilerParams(needs_layout_passes=False),
    )
    return f(input, loc)


# speedup vs baseline: 1.5649x; 1.0257x over previous
import jax
import jax.numpy as jnp
from jax import lax
from jax.experimental import pallas as pl
from jax.experimental.pallas import tpu as pltpu
from jax.experimental.pallas import tpu_sc as plsc

_B, _C, _L, _N = 16, 512, 4096, 1024
_GAMMA = 1.0
_CW = 16
_HC = 4
_NQ = _CW // _HC
_LANES = 16
_NG = _N // _LANES
_NSUB = 16
_ROWS = _N // _NSUB
_CCORE = _NSUB * _CW


def _sampler_body(inp, loc_in, out, loc_v, i0_v, w1_v, inb0, inb1,
                  outb, tmp, mrg, shm, sem0, sem1, seml, semp, semg, semo):
    cid = lax.axis_index("c")
    sid = lax.axis_index("s")
    wid = cid * _NSUB + sid
    c0 = wid * _CW

    def loc_dma(b, par):
        return pltpu.make_async_copy(
            loc_in.at[pl.ds(b * _N, _N)], loc_v.at[par], seml)

    def run_idx_loop(b, par):
        loc_dma(b, par).wait()

        @plsc.parallel_loop(0, _NG, unroll=2)
        def idx_body(j):
            loc = loc_v[par, pl.ds(j * _LANES, _LANES)]
            i0 = loc.astype(jnp.int32)
            i0_v[pl.ds(j * _LANES, _LANES)] = i0
            w1_v[pl.ds(j * _LANES, _LANES)] = loc - i0.astype(jnp.float32)

    def compute_quarter(buf, q, par):
        @plsc.parallel_loop(0, _NG, unroll=8)
        def grp_body(g):
            n_base = g * _LANES
            sl = pl.ds(n_base, _LANES)
            i0 = i0_v[sl]
            w1 = w1_v[sl]
            i1 = jnp.minimum(i0 + 1, _L - 1)
            w0 = 1.0 - w1
            b_idx = jnp.full((_LANES,), par, jnp.int32)
            s_idx = jnp.full((_LANES,), g // 4, jnp.int32)
            p_base = ((n_base % 64) + lax.iota(jnp.int32, _LANES)) * _CW + q * _HC
            for c in range(_HC):
                c_idx = jnp.full((_LANES,), c, jnp.int32)
                v0 = plsc.load_gather(buf, [c_idx, i0])
                v1 = plsc.load_gather(buf, [c_idx, i1])
                r = w0 * v0 + w1 * v1
                p = p_base + c
                plsc.store_scatter(outb, [b_idx, s_idx, p >> 7, p & 127], r)

    def run_ilv():
        @plsc.parallel_loop(0, _ROWS, unroll=2)
        def row_body(n):
            po = (n % 8) * _CW
            ph = n // 8
            for j in range(_NSUB):
                v = tmp[j, ph, pl.ds(po, _LANES)]
                plsc.store_scatter(
                    mrg, [jnp.full((_LANES,), n, jnp.int32),
                          j * _CW + lax.iota(jnp.int32, _LANES)], v)

    def in_slab(b, q):
        return inp.at[b, pl.ds(c0 + q * _HC, _HC)]

    def out_slab(b):
        return out.at[b, pl.ds(sid * _ROWS, _ROWS),
                      pl.ds(cid * _CCORE, _CCORE)]

    def put_dma(par):
        return pltpu.make_async_copy(outb.at[par], shm.at[sid], semp)

    def get_dma():
        return pltpu.make_async_copy(shm.at[:, sid], tmp, semg)

    bufs = (inb0, inb1)
    sems = (sem0, sem1)

    loc_dma(0, 0).start()
    pltpu.async_copy(in_slab(0, 0), inb0, sem0)

    def per_batch(b, _):
        par = b % 2
        parp = 1 - par

        @pl.when(b > 0)
        def _():
            put_dma(parp).start()

        @pl.when(b + 1 < _B)
        def _():
            loc_dma(b + 1, parp).start()

        with jax.named_scope("idx_phase"):
            run_idx_loop(b, par)

        for q in range(_NQ):
            if q + 1 < _NQ:
                pltpu.async_copy(in_slab(b, q + 1),
                                 bufs[(q + 1) % 2], sems[(q + 1) % 2])
            else:
                @pl.when(b + 1 < _B)
                def _():
                    pltpu.async_copy(in_slab(b + 1, 0), bufs[0], sems[0])

            with jax.named_scope("in_wait"):
                pltpu.make_async_copy(in_slab(b, q), bufs[q % 2],
                                      sems[q % 2]).wait()
            with jax.named_scope("gather"):
                compute_quarter(bufs[q % 2], q, par)

            if q == 1:
                with jax.named_scope("put_wait"):
                    @pl.when(b > 0)
                    def _():
                        put_dma(parp).wait()
                with jax.named_scope("mrg_bar1"):
                    plsc.subcore_barrier()
                with jax.named_scope("mrg_get"):
                    @pl.when(b > 0)
                    def _():
                        get_dma().start()

        @pl.when(b > 0)
        def _():
            with jax.named_scope("get_wait"):
                get_dma().wait()
            with jax.named_scope("out_wait"):
                @pl.when(b > 1)
                def _():
                    pltpu.make_async_copy(mrg, out_slab(b - 2), semo).wait()
            with jax.named_scope("mrg_ilv"):
                run_ilv()
            with jax.named_scope("out_dma"):
                pltpu.async_copy(mrg, out_slab(b - 1), semo)

        with jax.named_scope("mrg_bar2"):
            plsc.subcore_barrier()
        return 0

    lax.fori_loop(0, _B, per_batch, 0)

    pltpu.sync_copy(outb.at[(_B - 1) % 2], shm.at[sid])
    plsc.subcore_barrier()
    pltpu.sync_copy(shm.at[:, sid], tmp)
    pltpu.make_async_copy(mrg, out_slab(_B - 2), semo).wait()
    run_ilv()
    pltpu.sync_copy(mrg, out_slab(_B - 1))


def kernel(input, point, offset):
    loc = jnp.clip(point[:, :, 0] + _GAMMA * offset[:, :, 0], 0.0,
                   float(_L - 1)).reshape(_B * _N)
    mesh = plsc.VectorSubcoreMesh(core_axis_name="c", subcore_axis_name="s")
    f = pl.kernel(
        _sampler_body,
        out_type=jax.ShapeDtypeStruct((_B, _N, _C), jnp.float32),
        mesh=mesh,
        scratch_types=[
            pltpu.VMEM((2, _N), jnp.float32),
            pltpu.VMEM((_N,), jnp.int32),
            pltpu.VMEM((_N,), jnp.float32),
            pltpu.VMEM((_HC, _L), jnp.float32),
            pltpu.VMEM((_HC, _L), jnp.float32),
            pltpu.VMEM((2, _NSUB, 8, 128), jnp.float32),
            pltpu.VMEM((_NSUB, 8, 128), jnp.float32),
            pltpu.VMEM((_ROWS, _CCORE), jnp.float32),
            pltpu.VMEM_SHARED((_NSUB, _NSUB, 8, 128), jnp.float32),
            pltpu.SemaphoreType.DMA,
            pltpu.SemaphoreType.DMA,
            pltpu.SemaphoreType.DMA,
            pltpu.SemaphoreType.DMA,
            pltpu.SemaphoreType.DMA,
            pltpu.SemaphoreType.DMA,
        ],
        compiler_params=pltpu.CompilerParams(needs_layout_passes=False),
    )
    return f(input, loc)
